# Initial kernel scaffold; baseline (speedup 1.0000x reference)
import jax
import jax.numpy as jnp
from jax.experimental import pallas as pl


def kernel(inputs):
    raise NotImplementedError("write your pallas kernel here")


# speedup vs baseline: 7.1457x; 7.1457x over previous
import functools

import jax
import jax.numpy as jnp
from jax import lax
from jax.experimental import pallas as pl
from jax.experimental.pallas import tpu as pltpu
from jax.experimental.pallas import tpu_sc as plsc

BATCH = 32
SIZE = 14
PIX = SIZE * SIZE
CH = 768
NCHUNK = CH // 16
K_PIX = 10
K_CEN = 88
CENTER = (SIZE // 2) * SIZE + SIZE // 2
OUT_COLS = K_CEN + PIX * K_PIX
NW = 32
ROWS_W = BATCH * PIX // NW
PIX_G = 49
NGROUP = ROWS_W // PIX_G


def _sc_body(x_hbm, out_hbm, chunk_v, cen_v, out_v):
    wid = lax.axis_index("s") * 2 + lax.axis_index("c")
    row0 = wid * ROWS_W

    neg = jnp.full((16,), -jnp.inf, jnp.float32)

    pltpu.sync_copy(x_hbm.at[pl.ds(row0 + CENTER, 1)], cen_v)
    zeros16 = jnp.zeros((16,), jnp.int32)
    for p in range(6):
        def cbody(c, carry):
            tv, ti = carry
            v = cen_v[0, pl.ds(c * 16, 16)]
            gi = lax.iota(jnp.int32, 16) + c * 16
            va, gia = plsc.sort_key_val(v, gi, descending=False)
            nv = jnp.maximum(tv, va)
            ni = jnp.where(tv >= va, ti, gia)
            tv2, ti2 = plsc.sort_key_val(nv, ni, descending=True)
            return tv2, ti2
        tv, ti = lax.fori_loop(0, NCHUNK, cbody, (neg, zeros16))
        out_v[0, pl.ds(p * 16, 16)] = tv
        plsc.store_scatter(cen_v, [zeros16, ti], neg)

    for g in range(NGROUP):
        pltpu.sync_copy(x_hbm.at[pl.ds(row0 + g * PIX_G, PIX_G)], chunk_v)

        def pbody(p, _):
            T = neg
            for c in range(NCHUNK):
                v = chunk_v[p, pl.ds(c * 16, 16)]
                va = lax.sort(v, dimension=0)
                T, _unused = plsc.sort_key_val(
                    jnp.maximum(T, va), T, descending=True)
            off = K_CEN + (g * PIX_G + p) * K_PIX
            out_v[0, pl.ds(off, 16)] = T
            return 0

        lax.fori_loop(0, PIX_G, pbody, 0)

    pltpu.sync_copy(out_v.at[:, pl.ds(0, OUT_COLS)],
                    out_hbm.at[pl.ds(wid, 1)])


@jax.jit
def _run(xr):
    mesh = plsc.VectorSubcoreMesh(core_axis_name="c", subcore_axis_name="s")
    fn = pl.kernel(
        _sc_body,
        out_type=jax.ShapeDtypeStruct((BATCH, OUT_COLS), jnp.float32),
        mesh=mesh,
        scratch_types=[
            pltpu.VMEM((PIX_G, CH), jnp.float32),
            pltpu.VMEM((1, CH), jnp.float32),
            pltpu.VMEM((1, OUT_COLS + 16), jnp.float32),
        ],
        compiler_params=pltpu.CompilerParams(
            use_tc_tiling_on_sc=False, needs_layout_passes=False),
    )
    return fn(xr)


def kernel(inputs):
    xr = inputs.reshape(BATCH * PIX, CH)
    return _run(xr)


# speedup vs baseline: 13.5176x; 1.8917x over previous
import jax
import jax.numpy as jnp
from jax import lax
from jax.experimental import pallas as pl
from jax.experimental.pallas import tpu as pltpu
from jax.experimental.pallas import tpu_sc as plsc

BATCH = 32
SIZE = 14
PIX = SIZE * SIZE
CH = 768
NCHUNK = CH // 16
K_PIX = 10
K_CEN = 88
CENTER = (SIZE // 2) * SIZE + SIZE // 2
OUT_COLS = K_CEN + PIX * K_PIX
NW = 32
ROWS_W = BATCH * PIX // NW
PIX_G = 49
NGROUP = ROWS_W // PIX_G
NP = 7
NBLOCK = PIX_G // NP


def _sc_body(x_hbm, out_hbm, chunk_v, cen_v, out_v, dma_sem):
    wid = lax.axis_index("s") * 2 + lax.axis_index("c")
    g0 = wid * NGROUP

    neg = jnp.full((16,), -jnp.inf, jnp.float32)

    pltpu.sync_copy(
        x_hbm.at[pl.ds(g0 + CENTER // PIX_G, 1), pl.ds(CENTER % PIX_G, 1)],
        cen_v)
    zeros16 = jnp.zeros((16,), jnp.int32)
    for p in range(6):
        def cbody(c, carry):
            tv, ti = carry
            v = cen_v[0, 0, pl.ds(c * 16, 16)]
            gi = lax.iota(jnp.int32, 16) + c * 16
            va, gia = plsc.sort_key_val(v, gi, descending=False)
            nv = jnp.maximum(tv, va)
            ni = jnp.where(tv >= va, ti, gia)
            return tuple(plsc.sort_key_val(nv, ni, descending=True))
        tv, ti = lax.fori_loop(0, NCHUNK, cbody, (neg, zeros16))
        out_v[0, pl.ds(p * 16, 16)] = tv
        plsc.store_scatter(cen_v, [zeros16, zeros16, ti], neg)

    pltpu.async_copy(x_hbm.at[pl.ds(g0, 1)], chunk_v.at[pl.ds(0, 1)],
                     dma_sem)

    def gbody(g, _):
        bsel = lax.rem(g, 2)
        pltpu.make_async_copy(x_hbm.at[pl.ds(g0 + g, 1)],
                              chunk_v.at[pl.ds(bsel, 1)], dma_sem).wait()

        @pl.when(g < NGROUP - 1)
        def _prefetch():
            pltpu.async_copy(x_hbm.at[pl.ds(g0 + g + 1, 1)],
                             chunk_v.at[pl.ds(1 - bsel, 1)], dma_sem)

        def bbody(j, _):
            p0 = j * NP
            tops = [neg] * NP
            for c in range(NCHUNK):
                vas = [
                    lax.sort(chunk_v[bsel, p0 + q, pl.ds(c * 16, 16)],
                             dimension=0)
                    for q in range(NP)
                ]
                for q in range(NP):
                    tops[q], _u = plsc.sort_key_val(
                        jnp.maximum(tops[q], vas[q]), vas[q],
                        descending=True)
            for q in range(NP):
                off = K_CEN + (g * PIX_G + p0 + q) * K_PIX
                out_v[0, pl.ds(off, 16)] = tops[q]
            return 0

        lax.fori_loop(0, NBLOCK, bbody, 0)
        return 0

    lax.fori_loop(0, NGROUP, gbody, 0)

    pltpu.sync_copy(out_v.at[:, pl.ds(0, OUT_COLS)],
                    out_hbm.at[pl.ds(wid, 1)])


@jax.jit
def _run(xr):
    mesh = plsc.VectorSubcoreMesh(core_axis_name="c", subcore_axis_name="s")
    fn = pl.kernel(
        _sc_body,
        out_type=jax.ShapeDtypeStruct((BATCH, OUT_COLS), jnp.float32),
        mesh=mesh,
        scratch_types=[
            pltpu.VMEM((2, PIX_G, CH), jnp.float32),
            pltpu.VMEM((1, 1, CH), jnp.float32),
            pltpu.VMEM((1, OUT_COLS + 16), jnp.float32),
            pltpu.SemaphoreType.DMA,
        ],
        compiler_params=pltpu.CompilerParams(
            use_tc_tiling_on_sc=False, needs_layout_passes=False),
    )
    return fn(xr)


def kernel(inputs):
    xr = inputs.reshape(BATCH * NGROUP, PIX_G, CH)
    return _run(xr)


# speedup vs baseline: 18.0876x; 1.3381x over previous
import jax
import jax.numpy as jnp
from jax import lax
from jax.experimental import pallas as pl
from jax.experimental.pallas import tpu as pltpu
from jax.experimental.pallas import tpu_sc as plsc

BATCH = 32
SIZE = 14
PIX = SIZE * SIZE
CH = 768
NCHUNK = CH // 16
K_PIX = 10
K_CEN = 88
OUT_COLS = K_CEN + PIX * K_PIX
NP = 7


def _sc_body(x_hbm, out_hbm, chunk_v, cen_v, out_v, stage_v, dma_sem):
    wid = lax.axis_index("s") * 2 + lax.axis_index("c")

    neg = jnp.full((16,), -jnp.inf, jnp.float32)

    pltpu.sync_copy(x_hbm.at[pl.ds(wid, 1), pl.ds(SIZE // 2, 1)],
                    chunk_v.at[pl.ds(1, 1)])
    for c in range(NCHUNK):
        cen_v[pl.ds(c * 16, 16)] = chunk_v[1, 0, SIZE // 2,
                                           pl.ds(c * 16, 16)]
    pltpu.async_copy(x_hbm.at[pl.ds(wid, 1), pl.ds(0, 1)],
                     chunk_v.at[pl.ds(0, 1)], dma_sem)

    zeros16 = jnp.zeros((16,), jnp.int32)
    for p in range(6):
        def cbody(c, carry):
            tv, ti = carry
            v = cen_v[pl.ds(c * 16, 16)]
            gi = lax.iota(jnp.int32, 16) + c * 16
            va, gia = plsc.sort_key_val(v, gi, descending=False)
            nv = jnp.maximum(tv, va)
            ni = jnp.where(tv >= va, ti, gia)
            return tuple(plsc.sort_key_val(nv, ni, descending=True))
        tv, ti = lax.fori_loop(0, NCHUNK, cbody, (neg, zeros16))
        out_v[pl.ds(p * 16, 16)] = tv
        plsc.store_scatter(cen_v, [ti], neg)

    def rbody(i, _):
        bsel = lax.rem(i, 2)
        pltpu.make_async_copy(x_hbm.at[pl.ds(wid, 1), pl.ds(i, 1)],
                              chunk_v.at[pl.ds(bsel, 1)], dma_sem).wait()

        @pl.when(i < SIZE - 1)
        def _prefetch():
            pltpu.async_copy(x_hbm.at[pl.ds(wid, 1), pl.ds(i + 1, 1)],
                             chunk_v.at[pl.ds(1 - bsel, 1)], dma_sem)

        def bbody(j, _):
            p0 = j * NP
            tops = [neg] * NP
            for c in range(NCHUNK):
                vas = [
                    lax.sort(chunk_v[bsel, 0, p0 + q, pl.ds(c * 16, 16)],
                             dimension=0)
                    for q in range(NP)
                ]
                for q in range(NP):
                    tops[q], _u = plsc.sort_key_val(
                        jnp.maximum(tops[q], vas[q]), vas[q],
                        descending=True)
            for q in range(NP):
                off = K_CEN + (i * SIZE + p0 + q) * K_PIX
                out_v[pl.ds(off, 16)] = tops[q]
            return 0

        lax.fori_loop(0, SIZE // NP, bbody, 0)
        return 0

    lax.fori_loop(0, SIZE, rbody, 0)

    for r in range(16):
        for cc in range(8):
            stage_v[r, pl.ds(cc * 16, 16)] = out_v[
                pl.ds(r * 128 + cc * 16, 16)]
    pltpu.sync_copy(stage_v, out_hbm.at[pl.ds(wid * 16, 16)])


@jax.jit
def _run(x):
    mesh = plsc.VectorSubcoreMesh(core_axis_name="c", subcore_axis_name="s")
    fn = pl.kernel(
        _sc_body,
        out_type=jax.ShapeDtypeStruct((BATCH * 16, 128), jnp.float32),
        mesh=mesh,
        scratch_types=[
            pltpu.VMEM((2, 1, SIZE, CH), jnp.float32),
            pltpu.VMEM((CH,), jnp.float32),
            pltpu.VMEM((OUT_COLS + 16,), jnp.float32),
            pltpu.VMEM((16, 128), jnp.float32),
            pltpu.SemaphoreType.DMA,
        ],
        compiler_params=pltpu.CompilerParams(needs_layout_passes=False),
    )
    return fn(x)


def kernel(inputs):
    return _run(inputs).reshape(BATCH, OUT_COLS)


# speedup vs baseline: 21.4022x; 1.1833x over previous
import jax
import jax.numpy as jnp
from jax import lax
from jax.experimental import pallas as pl
from jax.experimental.pallas import tpu as pltpu
from jax.experimental.pallas import tpu_sc as plsc

BATCH = 32
SIZE = 14
PIX = SIZE * SIZE
CH = 768
NCHUNK = CH // 16
K_PIX = 10
K_CEN = 88
OUT_COLS = K_CEN + PIX * K_PIX
NP = 8
NBLOCK = BATCH // NP
NXTRA = PIX % 32


def _sc_body(x_hbm, out_pix, out_cen, chunk_v, cslab_v, cen_v, stage_v,
             cstage_v, dma_sem, out_sem):
    wid = lax.axis_index("s") * 2 + lax.axis_index("c")
    start = jnp.where(wid < NXTRA, wid * 7, NXTRA + wid * 6)
    count = jnp.where(wid < NXTRA, 7, 6)

    neg = jnp.full((16,), -jnp.inf, jnp.float32)

    pltpu.sync_copy(x_hbm.at[pl.ds(SIZE // 2, 1), pl.ds(SIZE // 2, 1)],
                    cslab_v)
    for c in range(NCHUNK):
        cen_v[pl.ds(c * 16, 16)] = cslab_v[0, 0, wid, pl.ds(c * 16, 16)]
    pltpu.async_copy(
        x_hbm.at[pl.ds(start // SIZE, 1), pl.ds(lax.rem(start, SIZE), 1)],
        chunk_v.at[pl.ds(0, 1)], dma_sem)

    zeros16 = jnp.zeros((16,), jnp.int32)
    for p in range(6):
        def cbody(c, carry):
            tv, ti = carry
            v = cen_v[pl.ds(c * 16, 16)]
            gi = lax.iota(jnp.int32, 16) + c * 16
            va, gia = plsc.sort_key_val(v, gi, descending=False)
            nv = jnp.maximum(tv, va)
            ni = jnp.where(tv >= va, ti, gia)
            return tuple(plsc.sort_key_val(nv, ni, descending=True))
        tv, ti = lax.fori_loop(0, NCHUNK, cbody, (neg, zeros16))
        cstage_v[0, 0, pl.ds(p * 16, 16)] = tv
        plsc.store_scatter(cen_v, [ti], neg)
    pltpu.sync_copy(cstage_v, out_cen.at[pl.ds(wid, 1)])

    def sbody(k, _):
        s = start + k
        bsel = lax.rem(k, 2)
        pltpu.make_async_copy(
            x_hbm.at[pl.ds(SIZE // 2, 1), pl.ds(SIZE // 2, 1)],
            chunk_v.at[pl.ds(bsel, 1)], dma_sem).wait()

        @pl.when(k < count - 1)
        def _prefetch():
            s1 = s + 1
            pltpu.async_copy(
                x_hbm.at[pl.ds(s1 // SIZE, 1), pl.ds(lax.rem(s1, SIZE), 1)],
                chunk_v.at[pl.ds(1 - bsel, 1)], dma_sem)

        @pl.when(k >= 2)
        def _drain():
            pltpu.make_async_copy(stage_v.at[pl.ds(bsel, 1)],
                                  out_pix.at[pl.ds(s, 1)], out_sem).wait()

        def bbody(j, _):
            b0 = j * NP
            tops = [neg] * NP
            for c in range(NCHUNK):
                vas = [
                    lax.sort(chunk_v[bsel, 0, b0 + q, pl.ds(c * 16, 16)],
                             dimension=0)
                    for q in range(NP)
                ]
                for q in range(NP):
                    tops[q], _u = plsc.sort_key_val(
                        jnp.maximum(tops[q], vas[q]), vas[q],
                        descending=True)
            for q in range(NP):
                stage_v[bsel, j, pl.ds(q * 16, 16)] = tops[q]
            return 0

        lax.fori_loop(0, NBLOCK, bbody, 0)
        pltpu.async_copy(stage_v.at[pl.ds(bsel, 1)], out_pix.at[pl.ds(s, 1)],
                         out_sem)
        return 0

    lax.fori_loop(0, count, sbody, 0)
    pltpu.make_async_copy(stage_v.at[pl.ds(0, 1)],
                          out_pix.at[pl.ds(start, 1)], out_sem).wait()
    pltpu.make_async_copy(stage_v.at[pl.ds(0, 1)],
                          out_pix.at[pl.ds(start, 1)], out_sem).wait()


@jax.jit
def _run(x):
    mesh = plsc.VectorSubcoreMesh(core_axis_name="c", subcore_axis_name="s")
    fn = pl.kernel(
        _sc_body,
        out_type=(
            jax.ShapeDtypeStruct((PIX, NBLOCK, 128), jnp.float32),
            jax.ShapeDtypeStruct((BATCH, 1, 128), jnp.float32),
        ),
        mesh=mesh,
        scratch_types=[
            pltpu.VMEM((2, 1, BATCH, CH), jnp.float32),
            pltpu.VMEM((1, 1, BATCH, CH), jnp.float32),
            pltpu.VMEM((CH,), jnp.float32),
            pltpu.VMEM((2, NBLOCK, 128), jnp.float32),
            pltpu.VMEM((1, 1, 128), jnp.float32),
            pltpu.SemaphoreType.DMA,
            pltpu.SemaphoreType.DMA,
        ],
        compiler_params=pltpu.CompilerParams(needs_layout_passes=False),
    )
    return fn(x)


def kernel(inputs):
    xt = inputs.transpose(1, 2, 0, 3)
    pix, cen = _run(xt)
    main = pix.reshape(PIX, BATCH, 16)[:, :, :K_PIX]
    main = main.transpose(1, 0, 2).reshape(BATCH, PIX * K_PIX)
    return jnp.concatenate([cen[:, 0, :K_CEN], main], axis=1)


# speedup vs baseline: 21.8366x; 1.0203x over previous
import jax
import jax.numpy as jnp
from jax import lax
from jax.experimental import pallas as pl
from jax.experimental.pallas import tpu as pltpu
from jax.experimental.pallas import tpu_sc as plsc

BATCH = 32
SIZE = 14
PIX = SIZE * SIZE
CH = 768
NCHUNK = CH // 16
K_PIX = 10
K_CEN = 88
OUT_COLS = K_CEN + PIX * K_PIX
NP = 8
NBLOCK = BATCH // NP
NXTRA = PIX % 32


def _sort_desc(v):
    return plsc.sort_key_val(v, v, descending=True)[0]


def _bitonic_desc(c):
    n = len(c)
    if n == 1:
        return [_sort_desc(c[0])]
    h = [jnp.maximum(c[i], c[i + n // 2]) for i in range(n // 2)]
    l = [jnp.minimum(c[i], c[i + n // 2]) for i in range(n // 2)]
    return _bitonic_desc(h) + _bitonic_desc(l)


def _merge(a, b, top_only=False):
    n = len(a)
    rb = [lax.rev(v, (0,)) for v in reversed(b)]
    c = [jnp.maximum(a[i], rb[i]) for i in range(n)]
    top = _bitonic_desc(c)
    if top_only:
        return top
    d = [jnp.minimum(a[i], rb[i]) for i in range(n)]
    return top + _bitonic_desc(d)


def _sc_body(x_hbm, out_pix, out_cen, chunk_v, cslab_v, stage_v,
             cstage_v, dma_sem, out_sem):
    wid = lax.axis_index("s") * 2 + lax.axis_index("c")
    start = jnp.where(wid < NXTRA, wid * 7, NXTRA + wid * 6)
    count = jnp.where(wid < NXTRA, 7, 6)

    neg = jnp.full((16,), -jnp.inf, jnp.float32)

    pltpu.sync_copy(x_hbm.at[pl.ds(SIZE // 2, 1), pl.ds(SIZE // 2, 1)],
                    cslab_v)
    pltpu.async_copy(
        x_hbm.at[pl.ds(start // SIZE, 1), pl.ds(lax.rem(start, SIZE), 1)],
        chunk_v.at[pl.ds(0, 1)], dma_sem)

    runs = [[_sort_desc(cslab_v[0, 0, wid, pl.ds(c * 16, 16)])]
            for c in range(NCHUNK)]
    while len(runs) > 6:
        runs = [_merge(runs[2 * i], runs[2 * i + 1])
                for i in range(len(runs) // 2)]
    t01 = _merge(runs[0], runs[1], top_only=True)
    t23 = _merge(runs[2], runs[3], top_only=True)
    t45 = _merge(runs[4], runs[5], top_only=True)
    t = _merge(_merge(t01, t23, top_only=True), t45, top_only=True)
    for p in range(6):
        cstage_v[0, 0, pl.ds(p * 16, 16)] = t[p]
    pltpu.sync_copy(cstage_v, out_cen.at[pl.ds(wid, 1)])

    def sbody(k, _):
        s = start + k
        bsel = lax.rem(k, 2)
        pltpu.make_async_copy(
            x_hbm.at[pl.ds(SIZE // 2, 1), pl.ds(SIZE // 2, 1)],
            chunk_v.at[pl.ds(bsel, 1)], dma_sem).wait()

        @pl.when(k < count - 1)
        def _prefetch():
            s1 = s + 1
            pltpu.async_copy(
                x_hbm.at[pl.ds(s1 // SIZE, 1), pl.ds(lax.rem(s1, SIZE), 1)],
                chunk_v.at[pl.ds(1 - bsel, 1)], dma_sem)

        @pl.when(k >= 2)
        def _drain():
            pltpu.make_async_copy(stage_v.at[pl.ds(bsel, 1)],
                                  out_pix.at[pl.ds(s, 1)], out_sem).wait()

        def bbody(j, _):
            b0 = j * NP
            tops = [neg] * NP
            for c in range(NCHUNK):
                vas = [
                    lax.sort(chunk_v[bsel, 0, b0 + q, pl.ds(c * 16, 16)],
                             dimension=0)
                    for q in range(NP)
                ]
                for q in range(NP):
                    tops[q], _u = plsc.sort_key_val(
                        jnp.maximum(tops[q], vas[q]), vas[q],
                        descending=True)
            for q in range(NP):
                stage_v[bsel, j, pl.ds(q * 16, 16)] = tops[q]
            return 0

        lax.fori_loop(0, NBLOCK, bbody, 0)
        pltpu.async_copy(stage_v.at[pl.ds(bsel, 1)], out_pix.at[pl.ds(s, 1)],
                         out_sem)
        return 0

    lax.fori_loop(0, count, sbody, 0)
    pltpu.make_async_copy(stage_v.at[pl.ds(0, 1)],
                          out_pix.at[pl.ds(start, 1)], out_sem).wait()
    pltpu.make_async_copy(stage_v.at[pl.ds(0, 1)],
                          out_pix.at[pl.ds(start, 1)], out_sem).wait()


@jax.jit
def _run(x):
    mesh = plsc.VectorSubcoreMesh(core_axis_name="c", subcore_axis_name="s")
    fn = pl.kernel(
        _sc_body,
        out_type=(
            jax.ShapeDtypeStruct((PIX, NBLOCK, 128), jnp.float32),
            jax.ShapeDtypeStruct((BATCH, 1, 128), jnp.float32),
        ),
        mesh=mesh,
        scratch_types=[
            pltpu.VMEM((2, 1, BATCH, CH), jnp.float32),
            pltpu.VMEM((1, 1, BATCH, CH), jnp.float32),
            pltpu.VMEM((2, NBLOCK, 128), jnp.float32),
            pltpu.VMEM((1, 1, 128), jnp.float32),
            pltpu.SemaphoreType.DMA,
            pltpu.SemaphoreType.DMA,
        ],
        compiler_params=pltpu.CompilerParams(needs_layout_passes=False),
    )
    return fn(x)


def kernel(inputs):
    xt = inputs.transpose(1, 2, 0, 3)
    pix, cen = _run(xt)
    main = pix.reshape(PIX, BATCH, 16)[:, :, :K_PIX]
    main = main.transpose(1, 0, 2).reshape(BATCH, PIX * K_PIX)
    return jnp.concatenate([cen[:, 0, :K_CEN], main], axis=1)
